# baseline (speedup 1.0000x reference)
import functools

import jax
import jax.numpy as jnp
from jax import lax
from jax.experimental import pallas as pl
from jax.experimental.pallas import tpu as pltpu
from jax.experimental.pallas import tpu_sc as plsc

_VOCAB = 100000
_EMBED = 128
_BATCH = 4096

_NC = 2
_NS = 16
_NW = _NC * _NS
_B_PER_W = _BATCH // _NW


@functools.lru_cache(maxsize=None)
def _make_sc_gather():
    mesh = plsc.VectorSubcoreMesh(core_axis_name="c", subcore_axis_name="s")

    @functools.partial(
        pl.kernel,
        mesh=mesh,
        out_type=jax.ShapeDtypeStruct((_BATCH, _EMBED), jnp.float32),
        scratch_types=[
            pltpu.VMEM((_B_PER_W,), jnp.int32),
            pltpu.VMEM((_B_PER_W, _EMBED), jnp.float32),
            pltpu.SemaphoreType.DMA,
        ],
    )
    def gather_kernel(idx_hbm, table_hbm, out_hbm, idx_v, rows_v, sem):
        wid = lax.axis_index("s") * _NC + lax.axis_index("c")
        base = wid * _B_PER_W
        pltpu.sync_copy(idx_hbm.at[pl.ds(base, _B_PER_W)], idx_v)
        pltpu.async_copy(table_hbm.at[idx_v], rows_v, sem).wait()
        pltpu.sync_copy(rows_v, out_hbm.at[pl.ds(base, _B_PER_W)])

    return gather_kernel


_N_TILE = 1024
_STEPS = (_VOCAB + _N_TILE - 1) // _N_TILE
_N_TAIL = _VOCAB - (_STEPS - 1) * _N_TILE
_NBUF = 2
_NSPLIT = 4
_ROWS = _BATCH // _NSPLIT


_M_CHUNK = 512


def _matmul_body(emb_ref, w_ref, b_ref, out_ref):
    w_bf = w_ref[...].astype(jnp.bfloat16)
    bias = b_ref[0]
    for mi in range(_BATCH // _M_CHUNK):
        sl = pl.ds(mi * _M_CHUNK, _M_CHUNK)
        acc = lax.dot_general(
            emb_ref[sl, :].astype(jnp.bfloat16),
            w_bf,
            (((1,), (1,)), ((), ())),
            preferred_element_type=jnp.float32,
        )
        out_ref[sl, :] = acc + bias


def _projection(embeds, W, b3d):
    return pl.pallas_call(
        _matmul_body,
        grid=(_STEPS,),
        in_specs=[
            pl.BlockSpec((_BATCH, _EMBED), lambda j: (0, 0)),
            pl.BlockSpec((_N_TILE, _EMBED), lambda j: (j, 0)),
            pl.BlockSpec((1, 1, _N_TILE), lambda j: (j, 0, 0)),
        ],
        out_specs=pl.BlockSpec((_BATCH, _N_TILE), lambda j: (0, j)),
        out_shape=jax.ShapeDtypeStruct((_BATCH, _VOCAB), jnp.float32),
        compiler_params=pltpu.CompilerParams(
            dimension_semantics=("arbitrary",),
        ),
    )(embeds, W, b3d)


def kernel(inputs, emb_table, W, b):
    embeds = _make_sc_gather()(inputs, emb_table)
    b_pad = jnp.pad(b, (0, _STEPS * _N_TILE - _VOCAB)).reshape(_STEPS, 1, _N_TILE)
    return _projection(embeds, W, b_pad)


# speedup vs baseline: 3.3721x; 3.3721x over previous
import functools

import jax
import jax.numpy as jnp
from jax import lax
from jax.experimental import pallas as pl
from jax.experimental.pallas import tpu as pltpu
from jax.experimental.pallas import tpu_sc as plsc

_VOCAB = 100000
_EMBED = 128
_BATCH = 4096

_NC = 2
_NS = 16
_NW = _NC * _NS
_B_PER_W = _BATCH // _NW


@functools.lru_cache(maxsize=None)
def _make_sc_gather():
    mesh = plsc.VectorSubcoreMesh(core_axis_name="c", subcore_axis_name="s")

    @functools.partial(
        pl.kernel,
        mesh=mesh,
        out_type=jax.ShapeDtypeStruct((_BATCH, _EMBED), jnp.float32),
        scratch_types=[
            pltpu.VMEM((_B_PER_W,), jnp.int32),
            pltpu.VMEM((_B_PER_W, _EMBED), jnp.float32),
            pltpu.SemaphoreType.DMA,
        ],
    )
    def gather_kernel(idx_hbm, table_hbm, out_hbm, idx_v, rows_v, sem):
        wid = lax.axis_index("s") * _NC + lax.axis_index("c")
        base = wid * _B_PER_W
        pltpu.sync_copy(idx_hbm.at[pl.ds(base, _B_PER_W)], idx_v)
        pltpu.async_copy(table_hbm.at[idx_v], rows_v, sem).wait()
        pltpu.sync_copy(rows_v, out_hbm.at[pl.ds(base, _B_PER_W)])

    return gather_kernel


_N_TILE = 1024
_STEPS = (_VOCAB + _N_TILE - 1) // _N_TILE


def _matmul_body(w_ref, emb_ref, b_ref, out_ref):
    acc = lax.dot_general(
        w_ref[...],
        emb_ref[...],
        (((1,), (1,)), ((), ())),
        preferred_element_type=jnp.float32,
    )
    out_ref[...] = acc + b_ref[...]


def _projection_t(W, embeds, b2d):
    return pl.pallas_call(
        _matmul_body,
        grid=(_STEPS,),
        in_specs=[
            pl.BlockSpec((_N_TILE, _EMBED), lambda j: (j, 0)),
            pl.BlockSpec((_BATCH, _EMBED), lambda j: (0, 0)),
            pl.BlockSpec((_N_TILE, 1), lambda j: (j, 0)),
        ],
        out_specs=pl.BlockSpec((_N_TILE, _BATCH), lambda j: (j, 0)),
        out_shape=jax.ShapeDtypeStruct((_VOCAB, _BATCH), jnp.float32),
        compiler_params=pltpu.CompilerParams(
            dimension_semantics=("arbitrary",),
        ),
    )(W, embeds, b2d)


def kernel(inputs, emb_table, W, b):
    embeds = _make_sc_gather()(inputs, emb_table)
    out_t = _projection_t(W, embeds, b.reshape(_VOCAB, 1))
    return out_t.T


# speedup vs baseline: 3.6820x; 1.0919x over previous
import functools

import jax
import jax.numpy as jnp
from jax import lax
from jax.experimental import pallas as pl
from jax.experimental.pallas import tpu as pltpu
from jax.experimental.pallas import tpu_sc as plsc

_VOCAB = 100000
_EMBED = 128
_BATCH = 4096

_NC = 2
_NS = 16
_NW = _NC * _NS
_B_PER_W = _BATCH // _NW


@functools.lru_cache(maxsize=None)
def _make_sc_gather():
    mesh = plsc.VectorSubcoreMesh(core_axis_name="c", subcore_axis_name="s")

    @functools.partial(
        pl.kernel,
        mesh=mesh,
        out_type=jax.ShapeDtypeStruct((_BATCH, _EMBED), jnp.float32),
        scratch_types=[
            pltpu.VMEM((_B_PER_W,), jnp.int32),
            pltpu.VMEM((_B_PER_W, _EMBED), jnp.float32),
            pltpu.SemaphoreType.DMA,
        ],
    )
    def gather_kernel(idx_hbm, table_hbm, out_hbm, idx_v, rows_v, sem):
        wid = lax.axis_index("s") * _NC + lax.axis_index("c")
        base = wid * _B_PER_W
        pltpu.sync_copy(idx_hbm.at[pl.ds(base, _B_PER_W)], idx_v)
        pltpu.async_copy(table_hbm.at[idx_v], rows_v, sem).wait()
        pltpu.sync_copy(rows_v, out_hbm.at[pl.ds(base, _B_PER_W)])

    return gather_kernel


_N_TILE = 1024
_STEPS = (_VOCAB + _N_TILE - 1) // _N_TILE


def _matmul_body(w_ref, emb_ref, b_ref, out_ref):
    acc = lax.dot_general(
        w_ref[...],
        emb_ref[...],
        (((1,), (1,)), ((), ())),
        preferred_element_type=jnp.float32,
    )
    out_ref[...] = acc + jnp.transpose(b_ref[0], (1, 0))


def _projection_t(W, embeds, b2d):
    return pl.pallas_call(
        _matmul_body,
        grid=(_STEPS,),
        in_specs=[
            pl.BlockSpec((_N_TILE, _EMBED), lambda j: (j, 0)),
            pl.BlockSpec((_BATCH, _EMBED), lambda j: (0, 0)),
            pl.BlockSpec((1, 1, _N_TILE), lambda j: (j, 0, 0)),
        ],
        out_specs=pl.BlockSpec((_N_TILE, _BATCH), lambda j: (j, 0)),
        out_shape=jax.ShapeDtypeStruct((_VOCAB, _BATCH), jnp.float32),
        compiler_params=pltpu.CompilerParams(
            dimension_semantics=("arbitrary",),
        ),
    )(W, embeds, b2d)


def kernel(inputs, emb_table, W, b):
    embeds = _make_sc_gather()(inputs, emb_table)
    b_pad = jnp.pad(b, (0, _STEPS * _N_TILE - _VOCAB)).reshape(_STEPS, 1, _N_TILE)
    out_t = _projection_t(W, embeds, b_pad)
    return out_t.T
